# baseline (speedup 1.0000x reference)
import jax
import jax.numpy as jnp
from jax.experimental import pallas as pl
from jax.experimental.pallas import tpu as pltpu

N = 1024
EMB = 12
P = 32
S = 512
R = 256
NB = N // R

_HI = jax.lax.Precision.HIGHEST
_DEF = jax.lax.Precision.DEFAULT


def _dot(a, b, prec=_DEF):
    return jax.lax.dot_general(a, b, (((1,), (0,)), ((), ())),
                               precision=prec,
                               preferred_element_type=jnp.float32)


def _dot_nt(a, b, prec=_DEF):
    return jax.lax.dot_general(a, b, (((1,), (1,)), ((), ())),
                               precision=prec,
                               preferred_element_type=jnp.float32)


def _fused_kernel(emb_ref, stops_r_ref, W_l_ref, W_r_ref, b_l_row_ref,
                  W_eT_ref, dist_ref, Wc_ref, bc_ref, be_ref, wv_ref,
                  out_ref,
                  xn_s, pref_s, u_s, act_s, v_s,
                  vacc_s, h_s, hr_s, colsum_s, act_row_s,
                  wces_s, wcd_s, const_s):
    i = pl.program_id(0)

    @pl.when(i == 0)
    def _init():
        emb = emb_ref[...]
        norm = jnp.sqrt(jnp.sum(emb * emb, axis=1, keepdims=True))
        xn_s[...] = emb / jnp.maximum(norm, 1e-8)
        row_iota = jax.lax.broadcasted_iota(jnp.int32, (N, 1), 0)
        oh = (row_iota == stops_r_ref[...]).astype(jnp.float32)
        act_s[...] = jnp.max(oh, axis=1, keepdims=True)
        ones_row = jnp.ones((1, S), jnp.float32)
        act_row_s[...] = jnp.minimum(_dot_nt(ones_row, oh), 1.0)
        We_r_row = W_eT_ref[:, P:]
        gl = _dot_nt(We_r_row, W_l_ref[...])
        gr = _dot_nt(We_r_row, W_r_ref[...])
        h_s[...] = _dot_nt(gl, emb)
        hr_s[...] = _dot_nt(gr, emb)
        vacc_s[...] = jnp.zeros_like(vacc_s)
        colsum_s[...] = jnp.zeros_like(colsum_s)
        const_s[...] = (bc_ref[...]
                        + wv_ref[0:1, 0:1] * Wc_ref[P + 2 * N:P + 2 * N + 1, :]
                        + wv_ref[0:1, 1:2] * Wc_ref[P + 2 * N + 1:P + 2 * N + 2, :])

    @pl.when(i == 1)
    def _cast_wc_es():
        wces_s[...] = Wc_ref[P:P + N, :].astype(jnp.bfloat16)

    @pl.when(i == 2)
    def _cast_wc_d():
        wcd_s[...] = Wc_ref[P + N:P + 2 * N, :].astype(jnp.bfloat16)

    @pl.when(i < NB)
    def _phase_a():
        emb_blk = emb_ref[pl.ds(i * R, R), :]
        xn_blk = xn_s[pl.ds(i * R, R), :]

        sim = _dot_nt(xn_blk, xn_s[...], _HI)

        row_g = i * R + jax.lax.broadcasted_iota(jnp.int32, (R, 1), 0)
        col_g = jax.lax.broadcasted_iota(jnp.int32, (1, N), 1)
        act_col = act_s[pl.ds(i * R, R), :] > 0.0
        valid = (row_g != col_g) & act_col & (act_row_s[...] > 0.0)
        A = jnp.where(valid & (sim > 0.5), 1.0, 0.0)

        deg_col = jnp.maximum(jnp.sum(A, axis=1, keepdims=True), 1.0)
        aggr = _dot(A, emb_ref[...]) / deg_col
        pref = (_dot(aggr, W_l_ref[...]) + _dot(emb_blk, W_r_ref[...])
                + b_l_row_ref[...])
        pref_s[pl.ds(i * R, R), :] = pref
        u_s[pl.ds(i * R, R), :] = _dot_nt(pref, W_eT_ref[:, :P])

        vacc_s[...] += _dot(h_s[:, pl.ds(i * R, R)], A)
        colsum_s[...] += jnp.sum(A, axis=0, keepdims=True)

        @pl.when(i == NB - 1)
        def _finish_a():
            deg_row = jnp.maximum(colsum_s[...], 1.0)
            c0 = jnp.sum(b_l_row_ref[...] * W_eT_ref[:, P:],
                         axis=1, keepdims=True)
            v_s[...] = vacc_s[...] / deg_row + hr_s[...] + c0

    @pl.when(i >= NB)
    def _phase_b():
        j = i - NB
        u = u_s[pl.ds(j * R, R), :]
        v = v_s[...]
        es = u + v + be_ref[...]
        es = jnp.where(es > 0, es, 0.01 * es)
        acc = _dot(es.astype(jnp.bfloat16), wces_s[...])
        acc += _dot(dist_ref[...].astype(jnp.bfloat16), wcd_s[...])
        acc += _dot(pref_s[pl.ds(j * R, R), :], Wc_ref[0:P, :])
        acc += const_s[...]
        acc += act_s[pl.ds(j * R, R), :] * Wc_ref[P + 2 * N + 2:P + 2 * N + 3, :]
        m = jnp.max(acc, axis=1, keepdims=True)
        sh = acc - m
        lse = jnp.log(jnp.sum(jnp.exp(sh), axis=1, keepdims=True))
        out_ref[...] = sh - lse


def kernel(edge_index, dist, stops, weekday, vehicles, emb,
           W_l, b_l, W_r, W_e, b_e, W_c, b_c):
    del edge_index
    f32 = jnp.float32
    stops_r = stops.reshape(1, S)
    W_eT = W_e.reshape(1, 2 * P).astype(f32)
    b_l_row = b_l.reshape(1, P).astype(f32)
    bc_row = b_c.reshape(1, N).astype(f32)
    be_11 = b_e.reshape(1, 1).astype(f32)
    wv = jnp.stack([jnp.asarray(weekday, f32).reshape(()),
                    jnp.asarray(vehicles, f32).reshape(())]).reshape(1, 2)

    const_spec = lambda shape: pl.BlockSpec(shape, lambda i: (0, 0))

    out = pl.pallas_call(
        _fused_kernel,
        grid=(2 * NB,),
        in_specs=[
            const_spec((N, EMB)),
            const_spec((1, S)),
            const_spec((EMB, P)), const_spec((EMB, P)),
            const_spec((1, P)), const_spec((1, 2 * P)),
            pl.BlockSpec((R, N), lambda i: (jnp.maximum(i - NB, 0), 0)),
            const_spec((P + 2 * N + 3, N)),
            const_spec((1, N)), const_spec((1, 1)), const_spec((1, 2)),
        ],
        out_specs=pl.BlockSpec((R, N), lambda i: (jnp.maximum(i - NB, 0), 0)),
        out_shape=jax.ShapeDtypeStruct((N, N), f32),
        scratch_shapes=[
            pltpu.VMEM((N, EMB), f32),
            pltpu.VMEM((N, P), f32),
            pltpu.VMEM((N, 1), f32),
            pltpu.VMEM((N, 1), f32),
            pltpu.VMEM((1, N), f32),
            pltpu.VMEM((1, N), f32),
            pltpu.VMEM((1, N), f32),
            pltpu.VMEM((1, N), f32),
            pltpu.VMEM((1, N), f32),
            pltpu.VMEM((1, N), f32),
            pltpu.VMEM((N, N), jnp.bfloat16),
            pltpu.VMEM((N, N), jnp.bfloat16),
            pltpu.VMEM((1, N), f32),
        ],
    )(emb.astype(f32), stops_r, W_l.astype(f32), W_r.astype(f32),
      b_l_row, W_eT, dist.astype(f32), W_c.astype(f32), bc_row, be_11, wv)
    return out


# speedup vs baseline: 1.1253x; 1.1253x over previous
import jax
import jax.numpy as jnp
from jax.experimental import pallas as pl
from jax.experimental.pallas import tpu as pltpu

N = 1024
EMB = 12
P = 32
S = 512
R = 256
NB = N // R

_HI = jax.lax.Precision.HIGHEST
_DEF = jax.lax.Precision.DEFAULT


def _dot(a, b, prec=_DEF):
    return jax.lax.dot_general(a, b, (((1,), (0,)), ((), ())),
                               precision=prec,
                               preferred_element_type=jnp.float32)


def _dot_nt(a, b, prec=_DEF):
    return jax.lax.dot_general(a, b, (((1,), (1,)), ((), ())),
                               precision=prec,
                               preferred_element_type=jnp.float32)


def _fused_kernel(emb_ref, stops_r_ref, W_l_ref, W_r_ref, b_l_row_ref,
                  W_eT_ref, dist_ref, Wc_ref, bc_ref, be_ref, wv_ref,
                  out_ref,
                  xn_s, pref_s, u_s, act_s, v_s,
                  vacc_s, h_s, hr_s, colsum_s, act_row_s,
                  const_s):
    i = pl.program_id(0)

    @pl.when(i == 0)
    def _init():
        emb = emb_ref[...]
        norm = jnp.sqrt(jnp.sum(emb * emb, axis=1, keepdims=True))
        xn = emb / jnp.maximum(norm, 1e-8)
        hi = xn.astype(jnp.bfloat16)
        r1 = xn - hi.astype(jnp.float32)
        mid = r1.astype(jnp.bfloat16)
        lo = (r1 - mid.astype(jnp.float32)).astype(jnp.bfloat16)
        xn_s[...] = jnp.concatenate([hi, mid, lo], axis=1)
        row_iota = jax.lax.broadcasted_iota(jnp.int32, (N, 1), 0)
        oh = (row_iota == stops_r_ref[...]).astype(jnp.float32)
        act_s[...] = jnp.max(oh, axis=1, keepdims=True)
        ones_row = jnp.ones((1, S), jnp.float32)
        act_row_s[...] = jnp.minimum(_dot_nt(ones_row, oh), 1.0)
        We_r_row = W_eT_ref[:, P:]
        gl = _dot_nt(We_r_row, W_l_ref[...])
        gr = _dot_nt(We_r_row, W_r_ref[...])
        h_s[...] = _dot_nt(gl, emb)
        hr_s[...] = _dot_nt(gr, emb)
        vacc_s[...] = jnp.zeros_like(vacc_s)
        colsum_s[...] = jnp.zeros_like(colsum_s)
        const_s[...] = (bc_ref[...]
                        + wv_ref[0:1, 0:1] * Wc_ref[P + 2 * N:P + 2 * N + 1, :]
                        + wv_ref[0:1, 1:2] * Wc_ref[P + 2 * N + 1:P + 2 * N + 2, :])

    @pl.when(i < NB)
    def _phase_a():
        emb_blk = emb_ref[pl.ds(i * R, R), :]
        xn_blk = xn_s[pl.ds(i * R, R), :]

        sim = _dot_nt(xn_blk, xn_s[...])

        row_g = i * R + jax.lax.broadcasted_iota(jnp.int32, (R, 1), 0)
        col_g = jax.lax.broadcasted_iota(jnp.int32, (1, N), 1)
        act_col = act_s[pl.ds(i * R, R), :] > 0.0
        valid = (row_g != col_g) & act_col & (act_row_s[...] > 0.0)
        A = jnp.where(valid & (sim > 0.5), 1.0, 0.0)

        deg_col = jnp.maximum(jnp.sum(A, axis=1, keepdims=True), 1.0)
        aggr = _dot(A, emb_ref[...]) / deg_col
        pref = (_dot(aggr, W_l_ref[...]) + _dot(emb_blk, W_r_ref[...])
                + b_l_row_ref[...])
        pref_s[pl.ds(i * R, R), :] = pref
        u_s[pl.ds(i * R, R), :] = _dot_nt(pref, W_eT_ref[:, :P])

        vacc_s[...] += _dot(h_s[:, pl.ds(i * R, R)], A)
        colsum_s[...] += jnp.sum(A, axis=0, keepdims=True)

        @pl.when(i == NB - 1)
        def _finish_a():
            deg_row = jnp.maximum(colsum_s[...], 1.0)
            c0 = jnp.sum(b_l_row_ref[...] * W_eT_ref[:, P:],
                         axis=1, keepdims=True)
            v_s[...] = vacc_s[...] / deg_row + hr_s[...] + c0

    @pl.when(i >= NB)
    def _phase_b():
        j = i - NB
        u = u_s[pl.ds(j * R, R), :]
        v = v_s[...]
        es = u + v + be_ref[...]
        es = jnp.where(es > 0, es, 0.01 * es)
        acc = _dot(es, Wc_ref[P:P + N, :])
        acc += _dot(dist_ref[...], Wc_ref[P + N:P + 2 * N, :])
        acc += _dot(pref_s[pl.ds(j * R, R), :], Wc_ref[0:P, :])
        acc += const_s[...]
        acc += act_s[pl.ds(j * R, R), :] * Wc_ref[P + 2 * N + 2:P + 2 * N + 3, :]
        m = jnp.max(acc, axis=1, keepdims=True)
        sh = acc - m
        lse = jnp.log(jnp.sum(jnp.exp(sh), axis=1, keepdims=True))
        out_ref[...] = sh - lse


def kernel(edge_index, dist, stops, weekday, vehicles, emb,
           W_l, b_l, W_r, W_e, b_e, W_c, b_c):
    del edge_index
    f32 = jnp.float32
    stops_r = stops.reshape(1, S)
    W_eT = W_e.reshape(1, 2 * P).astype(f32)
    b_l_row = b_l.reshape(1, P).astype(f32)
    bc_row = b_c.reshape(1, N).astype(f32)
    be_11 = b_e.reshape(1, 1).astype(f32)
    wv = jnp.stack([jnp.asarray(weekday, f32).reshape(()),
                    jnp.asarray(vehicles, f32).reshape(())]).reshape(1, 2)

    const_spec = lambda shape: pl.BlockSpec(shape, lambda i: (0, 0))

    out = pl.pallas_call(
        _fused_kernel,
        grid=(2 * NB,),
        in_specs=[
            const_spec((N, EMB)),
            const_spec((1, S)),
            const_spec((EMB, P)), const_spec((EMB, P)),
            const_spec((1, P)), const_spec((1, 2 * P)),
            pl.BlockSpec((R, N), lambda i: (jnp.maximum(i - NB, 0), 0)),
            const_spec((P + 2 * N + 3, N)),
            const_spec((1, N)), const_spec((1, 1)), const_spec((1, 2)),
        ],
        out_specs=pl.BlockSpec((R, N), lambda i: (jnp.maximum(i - NB, 0), 0)),
        out_shape=jax.ShapeDtypeStruct((N, N), f32),
        scratch_shapes=[
            pltpu.VMEM((N, 3 * EMB), jnp.bfloat16),
            pltpu.VMEM((N, P), f32),
            pltpu.VMEM((N, 1), f32),
            pltpu.VMEM((N, 1), f32),
            pltpu.VMEM((1, N), f32),
            pltpu.VMEM((1, N), f32),
            pltpu.VMEM((1, N), f32),
            pltpu.VMEM((1, N), f32),
            pltpu.VMEM((1, N), f32),
            pltpu.VMEM((1, N), f32),
            pltpu.VMEM((1, N), f32),
        ],
    )(emb.astype(f32), stops_r, W_l.astype(f32), W_r.astype(f32),
      b_l_row, W_eT, dist.astype(f32), W_c.astype(f32), bc_row, be_11, wv)
    return out
